# baseline (speedup 1.0000x reference)
import jax
import jax.numpy as jnp
from jax import lax
from jax.experimental import pallas as pl
from jax.experimental.pallas import tpu as pltpu
from jax.experimental.pallas import tpu_sc as plsc

NUM_CORES = 2
NUM_SUBCORES = 16
NW = NUM_CORES * NUM_SUBCORES

BATCH = 16384
HIST = 50
EMBED = 32
OUT_DIM = 20

BAGS_PER_W = BATCH // NW
G_BAGS = 32
N_GROUPS = BAGS_PER_W // G_BAGS


V_PACK = 251904
V_VIEW = V_PACK * 4


def _sc_body(idx_hbm, table_hbm, out_hbm,
             idx_v, rows_v, out_v, isems, gsems, osems):
    wid = lax.axis_index("s") * NUM_CORES + lax.axis_index("c")
    bag0 = wid * BAGS_PER_W
    table_r = table_hbm

    def stage_idx(g, slot):
        return pltpu.async_copy(
            idx_hbm.at[pl.ds(bag0 + g * G_BAGS, G_BAGS)],
            idx_v.at[slot], isems[slot])

    def fire_gather(p):
        return [pltpu.async_copy(
            table_r.at[idx_v.at[p, b]], rows_v.at[p, b], gsems[p])
            for b in range(G_BAGS)]

    def reduce_group(g, p):
        def bag(i, c):
            acc0 = jnp.zeros((16,), jnp.float32)
            acc1 = jnp.zeros((16,), jnp.float32)
            for r in range(HIST):
                acc0 = acc0 + rows_v[p, i, r, pl.ds(0, 16)]
                acc1 = acc1 + rows_v[p, i, r, pl.ds(16, 16)]
            out_v[p, i, pl.ds(0, 16)] = acc0 * (1.0 / HIST)
            out_v[p, i, pl.ds(16, 16)] = acc1 * (1.0 / HIST)
            return c
        lax.fori_loop(0, G_BAGS, bag, 0)
        return pltpu.async_copy(
            out_v.at[p],
            out_hbm.at[pl.ds(bag0 + g * G_BAGS, G_BAGS)], osems[p])

    h_idx = {0: stage_idx(0, 0)}
    h_gat = {}
    h_out = {}
    h_idx[0].wait()
    h_gat[0] = fire_gather(0)
    h_idx[1] = stage_idx(1, 1)
    for g in range(N_GROUPS):
        p = g % 2
        if g + 1 < N_GROUPS:
            h_idx[g + 1].wait()
            h_gat[g + 1] = fire_gather(1 - p)
        for h in h_gat[g]:
            h.wait()
        if g + 2 < N_GROUPS:
            h_idx[g + 2] = stage_idx(g + 2, p)
        if g - 2 >= 0:
            h_out[g - 2].wait()
        h_out[g] = reduce_group(g, p)
    h_out[N_GROUPS - 2].wait()
    h_out[N_GROUPS - 1].wait()


_sc_mesh = plsc.VectorSubcoreMesh(
    core_axis_name="c", subcore_axis_name="s",
    num_cores=NUM_CORES, num_subcores=NUM_SUBCORES)

def _repack_body(x_hbm, o_ref, x_v, sem):
    i = pl.program_id(0)
    copy = pltpu.make_async_copy(
        x_hbm.at[pl.ds(i * 8192, 8192)], x_v, sem)
    copy.start()
    copy.wait()
    x = x_v[...]
    parts = [lax.slice(x, (2048 * k, 0), (2048 * (k + 1), EMBED))
             for k in range(4)]
    o_ref[...] = jnp.concatenate(parts, axis=1)


def _tc_repack(table):
    return pl.pallas_call(
        _repack_body,
        grid=(122,),
        in_specs=[pl.BlockSpec(memory_space=pl.ANY)],
        out_specs=pl.BlockSpec((2048, 128), lambda i: (i, 0)),
        out_shape=jax.ShapeDtypeStruct((V_PACK, 128), jnp.float32),
        scratch_shapes=[
            pltpu.VMEM((8192, EMBED), jnp.float32),
            pltpu.SemaphoreType.DMA,
        ],
    )(table)


_sc_call = pl.kernel(
    _sc_body,
    out_type=jax.ShapeDtypeStruct((BATCH, EMBED), jnp.float32),
    mesh=_sc_mesh,
    scratch_types=[
        pltpu.VMEM((2, G_BAGS, HIST), jnp.int32),
        pltpu.VMEM((2, G_BAGS, HIST, EMBED), jnp.float32),
        pltpu.VMEM((2, G_BAGS, EMBED), jnp.float32),
        [pltpu.SemaphoreType.DMA, pltpu.SemaphoreType.DMA],
        [pltpu.SemaphoreType.DMA, pltpu.SemaphoreType.DMA],
        [pltpu.SemaphoreType.DMA, pltpu.SemaphoreType.DMA],
    ],
    compiler_params=pltpu.CompilerParams(use_tc_tiling_on_sc=False),
)


def _tc_body(bag_ref, w_ref, b_ref, o_ref):
    x = lax.dot_general(bag_ref[...], w_ref[...],
                        (((1,), (1,)), ((), ())),
                        preferred_element_type=jnp.float32)
    x = x + b_ref[...]
    x = x - jnp.max(x, axis=-1, keepdims=True)
    e = jnp.exp(x)
    o_ref[...] = e / jnp.sum(e, axis=-1, keepdims=True)


def _tc_call(bag, W, b):
    BB = 2048
    return pl.pallas_call(
        _tc_body,
        grid=(BATCH // BB,),
        in_specs=[
            pl.BlockSpec((BB, EMBED), lambda i: (i, 0)),
            pl.BlockSpec((OUT_DIM, EMBED), lambda i: (0, 0)),
            pl.BlockSpec((1, OUT_DIM), lambda i: (0, 0)),
        ],
        out_specs=pl.BlockSpec((BB, OUT_DIM), lambda i: (i, 0)),
        out_shape=jax.ShapeDtypeStruct((BATCH, OUT_DIM), jnp.float32),
    )(bag, W, b.reshape(1, OUT_DIM))


def kernel(input, table, W, b):
    i = input.astype(jnp.int32)
    idx = (i & ~jnp.int32(8191)) | ((i & 2047) << 2) | ((i >> 11) & 3)
    lin = _tc_repack(table)
    tail = lax.slice(table, (999424, 0), (1000001, EMBED))
    tail = jnp.pad(tail, ((0, 8192 - 577), (0, 0)))
    tail = jnp.concatenate(
        [tail[2048 * k:2048 * (k + 1)] for k in range(4)], axis=1)
    lin = lax.dynamic_update_slice(lin, tail, (249856, 0))
    bag = _sc_call(idx, jnp.reshape(lin, (V_VIEW, EMBED)))
    return _tc_call(bag, W, b)


# speedup vs baseline: 1.4221x; 1.4221x over previous
import jax
import jax.numpy as jnp
from jax import lax
from jax.experimental import pallas as pl
from jax.experimental.pallas import tpu as pltpu
from jax.experimental.pallas import tpu_sc as plsc

NUM_CORES = 2
NUM_SUBCORES = 16
NW = NUM_CORES * NUM_SUBCORES

BATCH = 16384
HIST = 50
EMBED = 32
OUT_DIM = 20

BAGS_PER_W = BATCH // NW
G_BAGS = 32
N_GROUPS = BAGS_PER_W // G_BAGS


def _sc_body(idx_hbm, table_hbm, out_hbm,
             idx_v, rows_v, out_v, isems, gsems, osems):
    wid = lax.axis_index("s") * NUM_CORES + lax.axis_index("c")
    bag0 = wid * BAGS_PER_W
    table_r = table_hbm

    def stage_idx(g, slot):
        return pltpu.async_copy(
            idx_hbm.at[pl.ds(bag0 + g * G_BAGS, G_BAGS)],
            idx_v.at[slot], isems[slot])

    def fire_gather(p):
        return [pltpu.async_copy(
            table_r.at[idx_v.at[p, b]], rows_v.at[p, b], gsems[p])
            for b in range(G_BAGS)]

    def reduce_group(g, p):
        def bag(i, c):
            acc0 = jnp.zeros((16,), jnp.float32)
            acc1 = jnp.zeros((16,), jnp.float32)
            for r in range(HIST):
                acc0 = acc0 + rows_v[p, i, r, pl.ds(0, 16)]
                acc1 = acc1 + rows_v[p, i, r, pl.ds(16, 16)]
            out_v[p, i, pl.ds(0, 16)] = acc0 * (1.0 / HIST)
            out_v[p, i, pl.ds(16, 16)] = acc1 * (1.0 / HIST)
            return c
        lax.fori_loop(0, G_BAGS, bag, 0)
        return pltpu.async_copy(
            out_v.at[p],
            out_hbm.at[pl.ds(bag0 + g * G_BAGS, G_BAGS)], osems[p])

    h_idx = {0: stage_idx(0, 0)}
    h_gat = {}
    h_out = {}
    h_idx[0].wait()
    h_gat[0] = fire_gather(0)
    h_idx[1] = stage_idx(1, 1)
    for g in range(N_GROUPS):
        p = g % 2
        if g + 1 < N_GROUPS:
            h_idx[g + 1].wait()
            h_gat[g + 1] = fire_gather(1 - p)
        for h in h_gat[g]:
            h.wait()
        if g + 2 < N_GROUPS:
            h_idx[g + 2] = stage_idx(g + 2, p)
        if g - 2 >= 0:
            h_out[g - 2].wait()
        h_out[g] = reduce_group(g, p)
    h_out[N_GROUPS - 2].wait()
    h_out[N_GROUPS - 1].wait()


_sc_mesh = plsc.VectorSubcoreMesh(
    core_axis_name="c", subcore_axis_name="s",
    num_cores=NUM_CORES, num_subcores=NUM_SUBCORES)

_sc_call = pl.kernel(
    _sc_body,
    out_type=jax.ShapeDtypeStruct((BATCH, EMBED), jnp.float32),
    mesh=_sc_mesh,
    scratch_types=[
        pltpu.VMEM((2, G_BAGS, HIST), jnp.int32),
        pltpu.VMEM((2, G_BAGS, HIST, EMBED), jnp.float32),
        pltpu.VMEM((2, G_BAGS, EMBED), jnp.float32),
        [pltpu.SemaphoreType.DMA, pltpu.SemaphoreType.DMA],
        [pltpu.SemaphoreType.DMA, pltpu.SemaphoreType.DMA],
        [pltpu.SemaphoreType.DMA, pltpu.SemaphoreType.DMA],
    ],
    compiler_params=pltpu.CompilerParams(use_tc_tiling_on_sc=False),
)


def _tc_body(bag_ref, w_ref, b_ref, o_ref):
    x = lax.dot_general(bag_ref[...], w_ref[...],
                        (((1,), (1,)), ((), ())),
                        preferred_element_type=jnp.float32)
    x = x + b_ref[...]
    x = x - jnp.max(x, axis=-1, keepdims=True)
    e = jnp.exp(x)
    o_ref[...] = e / jnp.sum(e, axis=-1, keepdims=True)


def _tc_call(bag, W, b):
    BB = 2048
    return pl.pallas_call(
        _tc_body,
        grid=(BATCH // BB,),
        in_specs=[
            pl.BlockSpec((BB, EMBED), lambda i: (i, 0)),
            pl.BlockSpec((OUT_DIM, EMBED), lambda i: (0, 0)),
            pl.BlockSpec((1, OUT_DIM), lambda i: (0, 0)),
        ],
        out_specs=pl.BlockSpec((BB, OUT_DIM), lambda i: (i, 0)),
        out_shape=jax.ShapeDtypeStruct((BATCH, OUT_DIM), jnp.float32),
    )(bag, W, b.reshape(1, OUT_DIM))


def kernel(input, table, W, b):
    idx = input.astype(jnp.int32)
    bag = _sc_call(idx, table)
    return _tc_call(bag, W, b)
